# initial kernel scaffold (unmeasured)
import jax
import jax.numpy as jnp
from jax import lax
from jax.experimental import pallas as pl
from jax.experimental.pallas import tpu as pltpu

N_DEV = 8
SQ = 2048
SKV = 2048
D_MODEL = 1024
H_LOC = 8
DH = 128
HD_LOC = H_LOC * DH
CHUNK = SQ // N_DEV
SCALE = 0.08838834764831843
BLK = 64


def _body(x_ref, wq_ref, k_ref, v_ref, wo_ref, out_ref,
          recv_ref, send_sems, recv_sems):
    my = lax.axis_index("i")
    right = (my + 1) % N_DEV

    q = jnp.dot(x_ref[:, :], wq_ref[:, :], preferred_element_type=jnp.float32)
    qb = lax.broadcasted_iota(jnp.int32, (SQ, SKV), 0) // BLK
    kb = lax.broadcasted_iota(jnp.int32, (SQ, SKV), 1) // BLK
    mask = kb <= qb
    acc = jnp.zeros((SQ, D_MODEL), jnp.float32)
    for h in range(H_LOC):
        sl = slice(h * DH, (h + 1) * DH)
        s = lax.dot_general(q[:, sl], k_ref[:, sl], (((1,), (1,)), ((), ())),
                            preferred_element_type=jnp.float32) * SCALE
        s = jnp.where(mask, s, -1e9)
        m = jnp.max(s, axis=1, keepdims=True)
        e = jnp.exp(s - m)
        w = e / jnp.sum(e, axis=1, keepdims=True)
        ctx = jnp.dot(w, v_ref[:, sl], preferred_element_type=jnp.float32)
        acc = acc + jnp.dot(ctx, wo_ref[sl, :],
                            preferred_element_type=jnp.float32)
    out_ref[:, :] = acc

    for t in range(N_DEV - 1):
        send_c = (my - t) % N_DEV
        recv_c = (my - t - 1) % N_DEV
        rdma = pltpu.make_async_remote_copy(
            src_ref=out_ref.at[pl.ds(send_c * CHUNK, CHUNK)],
            dst_ref=recv_ref.at[t],
            send_sem=send_sems.at[t],
            recv_sem=recv_sems.at[t],
            device_id=(right,),
            device_id_type=pl.DeviceIdType.MESH,
        )
        rdma.start()
        rdma.wait()
        out_ref[pl.ds(recv_c * CHUNK, CHUNK), :] += recv_ref[t]

    for t in range(N_DEV - 1):
        send_c = (my + 1 - t) % N_DEV
        rdma = pltpu.make_async_remote_copy(
            src_ref=out_ref.at[pl.ds(send_c * CHUNK, CHUNK)],
            dst_ref=out_ref.at[pl.ds(send_c * CHUNK, CHUNK)],
            send_sem=send_sems.at[N_DEV - 1 + t],
            recv_sem=recv_sems.at[N_DEV - 1 + t],
            device_id=(right,),
            device_id_type=pl.DeviceIdType.MESH,
        )
        rdma.start()
        rdma.wait()


def kernel(x, Wq, K_ext, V_ext, Wo):
    i = lax.axis_index("i")
    wq = lax.dynamic_slice(Wq, (0, i * HD_LOC), (D_MODEL, HD_LOC))
    wo = lax.dynamic_slice(Wo, (i * HD_LOC, 0), (HD_LOC, D_MODEL))
    x2 = x.reshape(SQ, D_MODEL)
    k2 = K_ext.reshape(SKV, HD_LOC)
    v2 = V_ext.reshape(SKV, HD_LOC)
    out = pl.pallas_call(
        _body,
        out_shape=jax.ShapeDtypeStruct((SQ, D_MODEL), jnp.float32),
        in_specs=[pl.BlockSpec(memory_space=pltpu.VMEM)] * 5,
        out_specs=pl.BlockSpec(memory_space=pltpu.VMEM),
        scratch_shapes=[
            pltpu.VMEM((N_DEV - 1, CHUNK, D_MODEL), jnp.float32),
            pltpu.SemaphoreType.DMA((2 * (N_DEV - 1),)),
            pltpu.SemaphoreType.DMA((2 * (N_DEV - 1),)),
        ],
        compiler_params=pltpu.CompilerParams(
            vmem_limit_bytes=128 * 1024 * 1024,
        ),
    )(x2, wq, k2, v2, wo)
    return out.reshape(1, SQ, D_MODEL)


# baseline (device time: 277374 ns/iter reference)
import jax
import jax.numpy as jnp
from jax import lax
from jax.experimental import pallas as pl
from jax.experimental.pallas import tpu as pltpu

N_DEV = 8
SQ = 2048
SKV = 2048
D_MODEL = 1024
H_LOC = 8
DH = 128
HD_LOC = H_LOC * DH
CHUNK = SQ // N_DEV
SCALE = 0.08838834764831843
BLK = 64


def _body(x_ref, wq_ref, k_ref, v_ref, wo_ref, out_ref,
          recv_ref, send_sems, recv_sems):
    my = lax.axis_index("i")
    right = (my + 1) % N_DEV

    for t in range(SQ // CHUNK):
        L = CHUNK * (t + 1)
        rows = slice(t * CHUNK, (t + 1) * CHUNK)
        q_t = jnp.dot(x_ref[rows, :], wq_ref[:, :],
                      preferred_element_type=jnp.float32)
        qb = (t * CHUNK + lax.broadcasted_iota(jnp.int32, (CHUNK, L), 0)) // BLK
        kb = lax.broadcasted_iota(jnp.int32, (CHUNK, L), 1) // BLK
        mask = kb <= qb
        ctx_list = []
        for h in range(H_LOC):
            sl = slice(h * DH, (h + 1) * DH)
            s = lax.dot_general(
                q_t[:, sl], k_ref[:L, sl], (((1,), (1,)), ((), ())),
                preferred_element_type=jnp.float32) * SCALE
            s = jnp.where(mask, s, -1e9)
            m = jnp.max(s, axis=1, keepdims=True)
            e = jnp.exp(s - m)
            w = e / jnp.sum(e, axis=1, keepdims=True)
            ctx_list.append(jnp.dot(w, v_ref[:L, sl],
                                    preferred_element_type=jnp.float32))
        ctx = jnp.concatenate(ctx_list, axis=1)
        out_ref[rows, :] = jnp.dot(ctx, wo_ref[:, :],
                                   preferred_element_type=jnp.float32)

    for t in range(N_DEV - 1):
        send_c = (my - t) % N_DEV
        recv_c = (my - t - 1) % N_DEV
        rdma = pltpu.make_async_remote_copy(
            src_ref=out_ref.at[pl.ds(send_c * CHUNK, CHUNK)],
            dst_ref=recv_ref.at[t],
            send_sem=send_sems.at[t],
            recv_sem=recv_sems.at[t],
            device_id=(right,),
            device_id_type=pl.DeviceIdType.MESH,
        )
        rdma.start()
        rdma.wait()
        out_ref[pl.ds(recv_c * CHUNK, CHUNK), :] += recv_ref[t]

    for t in range(N_DEV - 1):
        send_c = (my + 1 - t) % N_DEV
        rdma = pltpu.make_async_remote_copy(
            src_ref=out_ref.at[pl.ds(send_c * CHUNK, CHUNK)],
            dst_ref=out_ref.at[pl.ds(send_c * CHUNK, CHUNK)],
            send_sem=send_sems.at[N_DEV - 1 + t],
            recv_sem=recv_sems.at[N_DEV - 1 + t],
            device_id=(right,),
            device_id_type=pl.DeviceIdType.MESH,
        )
        rdma.start()
        rdma.wait()


def kernel(x, Wq, K_ext, V_ext, Wo):
    i = lax.axis_index("i")
    wq = lax.dynamic_slice(Wq, (0, i * HD_LOC), (D_MODEL, HD_LOC))
    wo = lax.dynamic_slice(Wo, (i * HD_LOC, 0), (HD_LOC, D_MODEL))
    x2 = x.reshape(SQ, D_MODEL)
    k2 = K_ext.reshape(SKV, HD_LOC)
    v2 = V_ext.reshape(SKV, HD_LOC)
    out = pl.pallas_call(
        _body,
        out_shape=jax.ShapeDtypeStruct((SQ, D_MODEL), jnp.float32),
        in_specs=[pl.BlockSpec(memory_space=pltpu.VMEM)] * 5,
        out_specs=pl.BlockSpec(memory_space=pltpu.VMEM),
        scratch_shapes=[
            pltpu.VMEM((N_DEV - 1, CHUNK, D_MODEL), jnp.float32),
            pltpu.SemaphoreType.DMA((2 * (N_DEV - 1),)),
            pltpu.SemaphoreType.DMA((2 * (N_DEV - 1),)),
        ],
        compiler_params=pltpu.CompilerParams(
            vmem_limit_bytes=128 * 1024 * 1024,
        ),
    )(x2, wq, k2, v2, wo)
    return out.reshape(1, SQ, D_MODEL)


# device time: 240177 ns/iter; 1.1549x vs baseline; 1.1549x over previous
import jax
import jax.numpy as jnp
from jax import lax
from jax.experimental import pallas as pl
from jax.experimental.pallas import tpu as pltpu

N_DEV = 8
SQ = 2048
SKV = 2048
D_MODEL = 1024
H_LOC = 8
DH = 128
HD_LOC = H_LOC * DH
CHUNK = SQ // N_DEV
SCALE = 0.08838834764831843
BLK = 64


def _body(x_ref, wq_ref, k_ref, v_ref, wo_ref, out_ref,
          recv_ref, send_sems, recv_sems):
    my = lax.axis_index("i")
    right = (my + 1) % N_DEV

    def compute_chunk(c):
        rows = pl.ds(c * CHUNK, CHUNK)
        q_t = jnp.dot(x_ref[rows, :], wq_ref[:, :],
                      preferred_element_type=jnp.float32)
        q_bf = q_t.astype(jnp.bfloat16)
        qb = (c * CHUNK
              + lax.broadcasted_iota(jnp.int32, (CHUNK, SKV), 0)) // BLK
        kb = lax.broadcasted_iota(jnp.int32, (CHUNK, SKV), 1) // BLK
        mask = kb <= qb
        ctx_list = []
        for h in range(H_LOC):
            sl = slice(h * DH, (h + 1) * DH)
            s = lax.dot_general(
                q_bf[:, sl], k_ref[:, sl], (((1,), (1,)), ((), ())),
                preferred_element_type=jnp.float32) * SCALE
            s = jnp.where(mask, s, -1e9)
            m = jnp.max(s, axis=1, keepdims=True)
            e = jnp.exp(s - m)
            w = (e / jnp.sum(e, axis=1, keepdims=True)).astype(jnp.bfloat16)
            ctx_list.append(jnp.dot(w, v_ref[:, sl],
                                    preferred_element_type=jnp.float32))
        ctx = jnp.concatenate(ctx_list, axis=1).astype(jnp.bfloat16)
        out_ref[rows, :] = jnp.dot(ctx, wo_ref[:, :],
                                   preferred_element_type=jnp.float32)

    compute_chunk(my)

    def rs_hop(t, carry):
        send_c = (my - t) % N_DEV
        next_c = (my - t - 1) % N_DEV
        rdma = pltpu.make_async_remote_copy(
            src_ref=out_ref.at[pl.ds(send_c * CHUNK, CHUNK)],
            dst_ref=recv_ref.at[t],
            send_sem=send_sems.at[t],
            recv_sem=recv_sems.at[t],
            device_id=(right,),
            device_id_type=pl.DeviceIdType.MESH,
        )
        rdma.start()
        compute_chunk(next_c)
        rdma.wait_recv()
        out_ref[pl.ds(next_c * CHUNK, CHUNK), :] += recv_ref[t]
        rdma.wait_send()
        return carry

    lax.fori_loop(0, N_DEV - 1, rs_hop, 0)

    def ag_hop(t, carry):
        send_c = (my + 1 - t) % N_DEV
        rdma = pltpu.make_async_remote_copy(
            src_ref=out_ref.at[pl.ds(send_c * CHUNK, CHUNK)],
            dst_ref=out_ref.at[pl.ds(send_c * CHUNK, CHUNK)],
            send_sem=send_sems.at[N_DEV - 1 + t],
            recv_sem=recv_sems.at[N_DEV - 1 + t],
            device_id=(right,),
            device_id_type=pl.DeviceIdType.MESH,
        )
        rdma.start()
        rdma.wait()
        return carry

    lax.fori_loop(0, N_DEV - 1, ag_hop, 0)


def kernel(x, Wq, K_ext, V_ext, Wo):
    i = lax.axis_index("i")
    wq = lax.dynamic_slice(Wq, (0, i * HD_LOC), (D_MODEL, HD_LOC))
    wo = lax.dynamic_slice(Wo, (i * HD_LOC, 0), (HD_LOC, D_MODEL))
    x2 = x.reshape(SQ, D_MODEL).astype(jnp.bfloat16)
    k2 = K_ext.reshape(SKV, HD_LOC).astype(jnp.bfloat16)
    v2 = V_ext.reshape(SKV, HD_LOC).astype(jnp.bfloat16)
    out = pl.pallas_call(
        _body,
        out_shape=jax.ShapeDtypeStruct((SQ, D_MODEL), jnp.float32),
        in_specs=[pl.BlockSpec(memory_space=pltpu.VMEM)] * 5,
        out_specs=pl.BlockSpec(memory_space=pltpu.VMEM),
        scratch_shapes=[
            pltpu.VMEM((N_DEV - 1, CHUNK, D_MODEL), jnp.float32),
            pltpu.SemaphoreType.DMA((2 * (N_DEV - 1),)),
            pltpu.SemaphoreType.DMA((2 * (N_DEV - 1),)),
        ],
        compiler_params=pltpu.CompilerParams(
            vmem_limit_bytes=128 * 1024 * 1024,
        ),
    )(x2, wq.astype(jnp.bfloat16), k2, v2, wo.astype(jnp.bfloat16))
    return out.reshape(1, SQ, D_MODEL)


# device time: 103568 ns/iter; 2.6782x vs baseline; 2.3190x over previous
import os

import jax
import jax.numpy as jnp
from jax import lax
from jax.experimental import pallas as pl
from jax.experimental.pallas import tpu as pltpu

_SKIP_RS = os.environ.get("DBG_SKIP_RS") == "1"
_SKIP_AG = os.environ.get("DBG_SKIP_AG") == "1"
_SKIP_COMPUTE = os.environ.get("DBG_SKIP_COMPUTE") == "1"

N_DEV = 8
SQ = 2048
SKV = 2048
D_MODEL = 1024
H_LOC = 8
DH = 128
HD_LOC = H_LOC * DH
SUB = 64
NQ = 4
PAIR = 512
SCALE = 0.08838834764831843
BLK = 64
NS = N_DEV - 1


def _body(x_ref, wq_ref, k_ref, v_ref, wo_ref, out_ref,
          rs_send_ref, rs_recv_ref, ag_send_ref, ag_recv_ref,
          send_sems, recv_sems):
    my = lax.axis_index("i")

    mask_diag = (
        lax.broadcasted_iota(jnp.int32, (PAIR, PAIR), 0) // BLK
        >= lax.broadcasted_iota(jnp.int32, (PAIR, PAIR), 1) // BLK)

    def compute_pair(p):
        r0 = p * PAIR
        L = (p + 1) * PAIR
        if _SKIP_COMPUTE:
            out_ref[r0:r0 + PAIR, :] = jnp.zeros((PAIR, D_MODEL),
                                                 jnp.float32)
            return
        q_t = jnp.dot(x_ref[r0:r0 + PAIR, :], wq_ref[:, :],
                      preferred_element_type=jnp.float32)
        q_bf = (q_t * SCALE).astype(jnp.bfloat16)
        ctx_list = []
        for h in range(H_LOC):
            sl = slice(h * DH, (h + 1) * DH)
            s = lax.dot_general(
                q_bf[:, sl], k_ref[:L, sl], (((1,), (1,)), ((), ())),
                preferred_element_type=jnp.float32)
            e_diag = jnp.where(mask_diag, jnp.exp(s[:, r0:]), 0.0)
            if r0:
                e = jnp.concatenate([jnp.exp(s[:, :r0]), e_diag], axis=1)
            else:
                e = e_diag
            denom = jnp.sum(e, axis=1, keepdims=True)
            ctx = jnp.dot(e.astype(jnp.bfloat16), v_ref[:L, sl],
                          preferred_element_type=jnp.float32) / denom
            ctx_list.append(ctx)
        ctx_all = jnp.concatenate(ctx_list, axis=1).astype(jnp.bfloat16)
        out_ref[r0:r0 + PAIR, :] = jnp.dot(
            ctx_all, wo_ref[:, :], preferred_element_type=jnp.float32)

    def send_sub(c):
        o = c % N_DEV
        q = c // N_DEV
        i_slot = ((o - my) % N_DEV) - 1 + NS * q
        j_slot = ((my - o) % N_DEV) - 1 + NS * q

        @pl.when(my != o)
        def _():
            rs_send_ref[pl.ds(i_slot, 1), :, :] = (
                out_ref[c * SUB:(c + 1) * SUB, :]
                .astype(jnp.bfloat16)[None])
            rdma = pltpu.make_async_remote_copy(
                src_ref=rs_send_ref.at[i_slot],
                dst_ref=rs_recv_ref.at[j_slot],
                send_sem=send_sems.at[i_slot],
                recv_sem=recv_sems.at[j_slot],
                device_id=(o,),
                device_id_type=pl.DeviceIdType.MESH,
            )
            rdma.start()

    def rs_wait(j, carry):
        rdma = pltpu.make_async_remote_copy(
            src_ref=rs_send_ref.at[0],
            dst_ref=rs_recv_ref.at[j],
            send_sem=send_sems.at[0],
            recv_sem=recv_sems.at[j],
            device_id=(my,),
            device_id_type=pl.DeviceIdType.MESH,
        )
        rdma.wait_recv()
        return carry

    def ag_wait(j, carry):
        rdma = pltpu.make_async_remote_copy(
            src_ref=ag_send_ref.at[0],
            dst_ref=ag_recv_ref.at[j],
            send_sem=send_sems.at[0],
            recv_sem=recv_sems.at[NQ * NS + j],
            device_id=(my,),
            device_id_type=pl.DeviceIdType.MESH,
        )
        rdma.wait_recv()
        q = j // NS
        src = (my + 1 + j % NS) % N_DEV
        sub = src + N_DEV * q
        out_ref[pl.ds(sub * SUB, SUB), :] = (
            ag_recv_ref[j].astype(jnp.float32))
        return carry

    def reduce_and_broadcast(q):
        rows = pl.ds((my + N_DEV * q) * SUB, SUB)
        lax.fori_loop(NS * q, NS * (q + 1), rs_wait, 0)
        lo = NS * q
        total = out_ref[rows, :] + jnp.sum(
            rs_recv_ref[lo:lo + NS, :, :].astype(jnp.float32), axis=0)
        out_ref[rows, :] = total
        ag_send_ref[q, :, :] = total.astype(jnp.bfloat16)
        if _SKIP_AG:
            return

        def ag_fire(i, carry):
            dest = (my + 1 + i) % N_DEV
            jd = ((my - dest) % N_DEV) - 1 + NS * q
            rdma = pltpu.make_async_remote_copy(
                src_ref=ag_send_ref.at[q],
                dst_ref=ag_recv_ref.at[jd],
                send_sem=send_sems.at[NQ * NS + NS * q + i],
                recv_sem=recv_sems.at[NQ * NS + jd],
                device_id=(dest,),
                device_id_type=pl.DeviceIdType.MESH,
            )
            rdma.start()
            return carry

        lax.fori_loop(0, NS, ag_fire, 0)

    for p in range(4):
        compute_pair(p)
        if not _SKIP_RS:
            for c in range(N_DEV * p, N_DEV * p + N_DEV):
                send_sub(c)
            if p >= 1:
                reduce_and_broadcast(p - 1)
    if not _SKIP_RS:
        reduce_and_broadcast(3)

    if not (_SKIP_RS or _SKIP_AG):
        lax.fori_loop(0, NQ * NS, ag_wait, 0)

    def drain(t, carry):
        rdma = pltpu.make_async_remote_copy(
            src_ref=rs_send_ref.at[0],
            dst_ref=rs_recv_ref.at[0],
            send_sem=send_sems.at[t],
            recv_sem=recv_sems.at[0],
            device_id=(my,),
            device_id_type=pl.DeviceIdType.MESH,
        )
        rdma.wait_send()
        return carry

    if not _SKIP_RS:
        lax.fori_loop(0, NQ * NS, drain, 0)
        if not _SKIP_AG:
            lax.fori_loop(NQ * NS, 2 * NQ * NS, drain, 0)


def kernel(x, Wq, K_ext, V_ext, Wo):
    i = lax.axis_index("i")
    wq = lax.dynamic_slice(Wq, (0, i * HD_LOC), (D_MODEL, HD_LOC))
    wo = lax.dynamic_slice(Wo, (i * HD_LOC, 0), (HD_LOC, D_MODEL))
    x2 = x.reshape(SQ, D_MODEL).astype(jnp.bfloat16)
    k2 = K_ext.reshape(SKV, HD_LOC).astype(jnp.bfloat16)
    v2 = V_ext.reshape(SKV, HD_LOC).astype(jnp.bfloat16)
    out = pl.pallas_call(
        _body,
        out_shape=jax.ShapeDtypeStruct((SQ, D_MODEL), jnp.float32),
        in_specs=[pl.BlockSpec(memory_space=pltpu.VMEM)] * 5,
        out_specs=pl.BlockSpec(memory_space=pltpu.VMEM),
        scratch_shapes=[
            pltpu.VMEM((NQ * NS, SUB, D_MODEL), jnp.bfloat16),
            pltpu.VMEM((NQ * NS, SUB, D_MODEL), jnp.bfloat16),
            pltpu.VMEM((NQ, SUB, D_MODEL), jnp.bfloat16),
            pltpu.VMEM((NQ * NS, SUB, D_MODEL), jnp.bfloat16),
            pltpu.SemaphoreType.DMA((2 * NQ * NS,)),
            pltpu.SemaphoreType.DMA((2 * NQ * NS,)),
        ],
        compiler_params=pltpu.CompilerParams(
            vmem_limit_bytes=128 * 1024 * 1024,
        ),
    )(x2, wq.astype(jnp.bfloat16), k2, v2, wo.astype(jnp.bfloat16))
    return out.reshape(1, SQ, D_MODEL)
